# baseline (speedup 1.0000x reference)
import functools

import jax
import jax.numpy as jnp
from jax import lax
from jax.experimental import pallas as pl
from jax.experimental.pallas import tpu as pltpu
from jax.experimental.pallas import tpu_sc as plsc

N = 10000
D = 128
S = 64
RD = 32
M = 200
LAM = 0.005
CUT = 6.0
E = 160000

NC = 2
NS = 16
NW = NC * NS

EPAD = 160256
EW = EPAD // NW
NPAD = 10112
NROW = NPAD // NS

_HD = D // 2

C0 = 0.28209479177387814
C1 = 0.4886025119029199
RBF_STEP = CUT / (RD - 1)
RBF_DEN = 1.0 / (2.0 * (CUT / RD) * (CUT / RD))


def _iota16():
    return lax.iota(jnp.int32, 16)


_EF_K = 512
_EF_FULL = EW // _EF_K
_EF_TAIL = EW - _EF_FULL * _EF_K


def _efront_body(pos_hbm, src_hbm, dst_hbm, h_hbm, vec_hbm, hsrc_hbm,
                 pos_v, src_v, dst_v, vec_v, rows_v, sem):
    wid = lax.axis_index("s") * NC + lax.axis_index("c")
    base = wid * EW
    pltpu.sync_copy(pos_hbm, pos_v)

    def chunk(start, k):
        pltpu.sync_copy(src_hbm.at[pl.ds(start, k)], src_v.at[pl.ds(0, k)])
        pltpu.sync_copy(dst_hbm.at[pl.ds(start, k)], dst_v.at[pl.ds(0, k)])
        cp = pltpu.async_copy(h_hbm.at[src_v], rows_v, sem)

        def inner(i, _):
            sv = src_v[pl.ds(i * 16, 16)] * 3
            dv = dst_v[pl.ds(i * 16, 16)] * 3
            lanes = (i * 16 + _iota16()) * 4
            acc = jnp.full((16,), 1e-12, jnp.float32)
            for comp in range(3):
                c16 = jnp.full((16,), comp, jnp.int32)
                ps = plsc.load_gather(pos_v, [sv + c16])
                pd = plsc.load_gather(pos_v, [dv + c16])
                vc = ps - pd
                acc = acc + vc * vc
                plsc.store_scatter(vec_v, [lanes + c16], vc)
            plsc.store_scatter(vec_v, [lanes + jnp.full((16,), 3, jnp.int32)], acc)
            return _

        lax.fori_loop(0, k // 16, inner, None)
        cp.wait()
        pltpu.sync_copy(rows_v.at[pl.ds(0, k), :], hsrc_hbm.at[pl.ds(start, k), :])
        pltpu.sync_copy(vec_v.at[pl.ds(0, k * 4)], vec_hbm.at[pl.ds(start * 4, k * 4)])

    def full_chunk(t, _):
        chunk(base + t * _EF_K, _EF_K)
        return _

    lax.fori_loop(0, _EF_FULL, full_chunk, None)
    chunk(base + _EF_FULL * _EF_K, _EF_TAIL)


def _efront(pos_pad, srcp, dstp, h):
    mesh = plsc.VectorSubcoreMesh(core_axis_name="c", subcore_axis_name="s")
    return pl.kernel(
        _efront_body,
        out_type=(
            jax.ShapeDtypeStruct((EPAD * 4,), jnp.float32),
            jax.ShapeDtypeStruct((EPAD, D), jnp.float32),
        ),
        mesh=mesh,
        scratch_types=[
            pltpu.VMEM(((N + 8) * 3,), jnp.float32),
            pltpu.VMEM((_EF_K,), jnp.int32),
            pltpu.VMEM((_EF_K,), jnp.int32),
            pltpu.VMEM((_EF_K * 4,), jnp.float32),
            pltpu.VMEM((_EF_K, D), jnp.float32),
            pltpu.SemaphoreType.DMA,
        ],
        compiler_params=pltpu.CompilerParams(needs_layout_passes=False, use_tc_tiling_on_sc=False),
    )(pos_pad.reshape((N + 8) * 3), srcp, dstp, h)


_W_TILE = 4096


def _wmsg_body(vec_ref, hsrc_ref, rbfc_ref, wfc_ref, bfc_ref, wsh_ref, out_ref):
    vec = vec_ref[...]
    x = vec[:, 0:1]
    y = vec[:, 1:2]
    z = vec[:, 2:3]
    ln = jnp.sqrt(vec[:, 3:4])
    lden = ln + 1e-9
    dd = ln - rbfc_ref[...]
    ef = jnp.exp(-(dd * dd) / (2.0 * (CUT / RD) * (CUT / RD)))
    w = jnp.dot(ef, wfc_ref[...], preferred_element_type=jnp.float32) + bfc_ref[...]
    sh = jnp.concatenate(
        [jnp.full((_W_TILE, 1), C0, jnp.float32), C1 * (x / lden), C1 * (y / lden),
         C1 * (z / lden)], axis=1)
    sm = jnp.dot(sh, wsh_ref[...], preferred_element_type=jnp.float32)
    res = hsrc_ref[...] * w * sm
    out_ref[0] = res[:, :_HD]
    out_ref[1] = res[:, _HD:]


def _wmsg(vecln, hsrc, rbf_c, Wfc, bfc, Wsh):
    grid = (pl.cdiv(EPAD, _W_TILE),)
    return pl.pallas_call(
        _wmsg_body,
        grid=grid,
        in_specs=[
            pl.BlockSpec((_W_TILE, 4), lambda i: (i, 0)),
            pl.BlockSpec((_W_TILE, D), lambda i: (i, 0)),
            pl.BlockSpec((1, RD), lambda i: (0, 0)),
            pl.BlockSpec((RD, D), lambda i: (0, 0)),
            pl.BlockSpec((1, D), lambda i: (0, 0)),
            pl.BlockSpec((4, D), lambda i: (0, 0)),
        ],
        out_specs=pl.BlockSpec((2, _W_TILE, _HD), lambda i: (0, i, 0)),
        out_shape=jax.ShapeDtypeStruct((2, EPAD, _HD), jnp.float32),
    )(vecln, hsrc, rbf_c.reshape(1, RD), Wfc, bfc.reshape(1, D), Wsh)


_ES_K = 512
_ES_PER = EPAD // NS
_ES_FULL = _ES_PER // _ES_K
_ES_TAIL = _ES_PER - _ES_FULL * _ES_K


def _escatter_body(dst_hbm, msg_hbm, zeros_hbm, out_hbm,
                   agg_sh, dst_v, rows_v, dst_vt, rows_vt):
    c = lax.axis_index("c")
    s = lax.axis_index("s")
    pltpu.sync_copy(zeros_hbm.at[pl.ds(s * NROW, NROW), :],
                    agg_sh.at[pl.ds(s * NROW, NROW), :])
    plsc.subcore_barrier()
    base = s * _ES_PER

    def full_chunk(t, _):
        start = base + t * _ES_K
        pltpu.sync_copy(dst_hbm.at[pl.ds(start, _ES_K)], dst_v)
        pltpu.sync_copy(msg_hbm.at[c, pl.ds(start, _ES_K), :], rows_v)
        pltpu.sync_copy(rows_v, agg_sh.at[dst_v], add=True)
        return _

    lax.fori_loop(0, _ES_FULL, full_chunk, None)
    start = base + _ES_FULL * _ES_K
    pltpu.sync_copy(dst_hbm.at[pl.ds(start, _ES_TAIL)], dst_vt)
    pltpu.sync_copy(msg_hbm.at[c, pl.ds(start, _ES_TAIL), :], rows_vt)
    pltpu.sync_copy(rows_vt, agg_sh.at[dst_vt], add=True)
    plsc.subcore_barrier()
    pltpu.sync_copy(agg_sh.at[pl.ds(s * NROW, NROW), :],
                    out_hbm.at[c, pl.ds(s * NROW, NROW), :])


def _escatter(dstp, msg, zeros_half):
    mesh = plsc.VectorSubcoreMesh(core_axis_name="c", subcore_axis_name="s")
    return pl.kernel(
        _escatter_body,
        out_type=jax.ShapeDtypeStruct((2, NPAD, _HD), jnp.float32),
        mesh=mesh,
        scratch_types=[
            pltpu.VMEM_SHARED((NPAD, _HD), jnp.float32),
            pltpu.VMEM((_ES_K,), jnp.int32),
            pltpu.VMEM((_ES_K, _HD), jnp.float32),
            pltpu.VMEM((_ES_TAIL,), jnp.int32),
            pltpu.VMEM((_ES_TAIL, _HD), jnp.float32),
        ],
        compiler_params=pltpu.CompilerParams(needs_layout_passes=False, use_tc_tiling_on_sc=False),
    )(dstp, msg, zeros_half)


ADJW = 208
ADJSZ = ADJW * ADJW
_AJ_SLAB = ADJSZ // NS
_AJ_K = 512
_AJ_FULL = EW // _AJ_K
_AJ_TAIL = EW - _AJ_FULL * _AJ_K
NRANK = NPAD


def _eadj_body(src_hbm, dst_hbm, mi_hbm, hl_hbm, posw_hbm,
               adj_hbm, hm_hbm, pm_hbm,
               rank_v, src_v, dst_v, adj_v, acc_v, mrg_v, mi_v, mi_g, hm_v, pm_v,
               stage_sh, sem):
    c = lax.axis_index("c")
    s = lax.axis_index("s")
    wid = s * NC + c
    pltpu.sync_copy(mi_hbm, mi_v)

    neg1 = jnp.full((16,), -1, jnp.int32)

    def initrank(i, _):
        rank_v[pl.ds(i * 16, 16)] = neg1
        return _

    lax.fori_loop(0, NRANK // 16, initrank, None)

    zero16 = jnp.zeros((16,), jnp.float32)

    def initadj(i, _):
        adj_v[pl.ds(i * 16, 16)] = zero16
        return _

    lax.fori_loop(0, ADJSZ // 16, initadj, None)

    def setrank(i, _):
        mi16 = mi_v[pl.ds(i * 16, 16)]
        plsc.store_scatter(rank_v, [mi16], i * 16 + _iota16())
        return _

    lax.fori_loop(0, ADJW // 16, setrank, None)

    base = wid * EW
    ones16 = jnp.ones((16,), jnp.float32)

    def chunk(start, k):
        pltpu.sync_copy(src_hbm.at[pl.ds(start, k)], src_v.at[pl.ds(0, k)])
        pltpu.sync_copy(dst_hbm.at[pl.ds(start, k)], dst_v.at[pl.ds(0, k)])

        def inner(i, _):
            sv = src_v[pl.ds(i * 16, 16)]
            dv = dst_v[pl.ds(i * 16, 16)]
            rs = plsc.load_gather(rank_v, [sv])
            rd = plsc.load_gather(rank_v, [dv])
            valid = (rs >= 0) & (rd >= 0)
            idx = jnp.where(valid, rs * ADJW + rd, 0)
            plsc.store_scatter(adj_v, [idx], ones16, mask=valid)
            return _

        lax.fori_loop(0, k // 16, inner, None)

    def full_chunk(t, _):
        chunk(base + t * _AJ_K, _AJ_K)
        return _

    lax.fori_loop(0, _AJ_FULL, full_chunk, None)
    chunk(base + _AJ_FULL * _AJ_K, _AJ_TAIL)

    pltpu.sync_copy(adj_v, stage_sh.at[pl.ds(s * ADJSZ, ADJSZ)])
    plsc.subcore_barrier()

    pltpu.sync_copy(stage_sh.at[pl.ds(s * _AJ_SLAB, _AJ_SLAB)], acc_v)
    for t in range(1, NS):
        pltpu.sync_copy(stage_sh.at[pl.ds(t * ADJSZ + s * _AJ_SLAB, _AJ_SLAB)], mrg_v)

        def madd(i, _):
            acc_v[pl.ds(i * 16, 16)] = acc_v[pl.ds(i * 16, 16)] + mrg_v[pl.ds(i * 16, 16)]
            return _

        lax.fori_loop(0, _AJ_SLAB // 16, madd, None)
    pltpu.sync_copy(acc_v, adj_hbm.at[c, pl.ds(s * _AJ_SLAB, _AJ_SLAB)])

    @pl.when((c == 0) & (s == 0))
    def _():
        pltpu.sync_copy(mi_hbm.at[pl.ds(0, ADJW // 2)], mi_g)
        pltpu.async_copy(hl_hbm.at[mi_g], hm_v, sem).wait()
        pltpu.sync_copy(hm_v, hm_hbm.at[pl.ds(0, ADJW // 2), :])
        pltpu.sync_copy(mi_hbm.at[pl.ds(ADJW // 2, ADJW // 2)], mi_g)
        pltpu.async_copy(hl_hbm.at[mi_g], hm_v, sem).wait()
        pltpu.sync_copy(hm_v, hm_hbm.at[pl.ds(ADJW // 2, ADJW // 2), :])

    @pl.when((c == 1) & (s == 0))
    def _():
        pltpu.async_copy(posw_hbm.at[mi_v], pm_v, sem).wait()
        pltpu.sync_copy(pm_v, pm_hbm)


def _eadj(srcp, dstp, mi_pad, h_local, posw):
    mesh = plsc.VectorSubcoreMesh(core_axis_name="c", subcore_axis_name="s")
    return pl.kernel(
        _eadj_body,
        out_type=(
            jax.ShapeDtypeStruct((2, ADJSZ), jnp.float32),
            jax.ShapeDtypeStruct((ADJW, D), jnp.float32),
            jax.ShapeDtypeStruct((ADJW, 16), jnp.float32),
        ),
        mesh=mesh,
        scratch_types=[
            pltpu.VMEM((NRANK,), jnp.int32),
            pltpu.VMEM((_AJ_K,), jnp.int32),
            pltpu.VMEM((_AJ_K,), jnp.int32),
            pltpu.VMEM((ADJSZ,), jnp.float32),
            pltpu.VMEM((_AJ_SLAB,), jnp.float32),
            pltpu.VMEM((_AJ_SLAB,), jnp.float32),
            pltpu.VMEM((ADJW,), jnp.int32),
            pltpu.VMEM((ADJW // 2,), jnp.int32),
            pltpu.VMEM((ADJW // 2, D), jnp.float32),
            pltpu.VMEM((ADJW, 16), jnp.float32),
            pltpu.VMEM_SHARED((NS * ADJSZ,), jnp.float32),
            pltpu.SemaphoreType.DMA,
        ],
        compiler_params=pltpu.CompilerParams(needs_layout_passes=False, use_tc_tiling_on_sc=False),
    )(srcp, dstp, mi_pad, h_local, posw)


_N_TILE = 400


def _nlocal_body(agg0_ref, agg1_ref, h_ref, wout_ref, bout_ref,
                 wm1_ref, bm1_ref, wm2_ref, bm2_ref, hl_ref, m_ref):
    agg = jnp.concatenate([agg0_ref[0], agg1_ref[0]], axis=1)
    hl = (jnp.dot(agg, wout_ref[...], preferred_element_type=jnp.float32)
          + bout_ref[...] + h_ref[...])
    hl_ref[...] = hl
    t = jnp.maximum(jnp.dot(hl[:, :S], wm1_ref[...], preferred_element_type=jnp.float32)
                    + bm1_ref[...], 0.0)
    logit = jnp.dot(t, wm2_ref[...], preferred_element_type=jnp.float32) + bm2_ref[...]
    m_ref[...] = 1.0 / (1.0 + jnp.exp(-logit))


def _nlocal(agg3, h, Wout, bout, Wm1, bm1, Wm2, bm2):
    grid = (N // _N_TILE,)
    return pl.pallas_call(
        _nlocal_body,
        grid=grid,
        in_specs=[
            pl.BlockSpec((1, _N_TILE, _HD), lambda i: (0, i, 0)),
            pl.BlockSpec((1, _N_TILE, _HD), lambda i: (1, i, 0)),
            pl.BlockSpec((_N_TILE, D), lambda i: (i, 0)),
            pl.BlockSpec((D, D), lambda i: (0, 0)),
            pl.BlockSpec((1, D), lambda i: (0, 0)),
            pl.BlockSpec((S, D), lambda i: (0, 0)),
            pl.BlockSpec((1, D), lambda i: (0, 0)),
            pl.BlockSpec((D, 1), lambda i: (0, 0)),
            pl.BlockSpec((1, 1), lambda i: (0, 0)),
        ],
        out_specs=[
            pl.BlockSpec((_N_TILE, D), lambda i: (i, 0)),
            pl.BlockSpec((_N_TILE, 1), lambda i: (i, 0)),
        ],
        out_shape=[
            jax.ShapeDtypeStruct((N, D), jnp.float32),
            jax.ShapeDtypeStruct((N, 1), jnp.float32),
        ],
    )(agg3, agg3, h, Wout, bout.reshape(1, D), Wm1, bm1.reshape(1, D),
      Wm2, bm2.reshape(1, 1))


_MB = 8


def _master_body(hm_ref, pm_ref, pmt_ref, adj_ref, adjt_ref, rbfc_ref, wq_ref, wk_ref,
                 wa1_ref, ba1_ref, wa2_ref, ba2_ref, wfc_ref, bfc_ref,
                 wsh_ref, wout_ref, bout_ref, av_ref, hh_ref):
    hm = hm_ref[...]
    hms = hm[:, :S]
    adj = adj_ref[...]
    adjt = adjt_ref[...]

    pm = pm_ref[...]
    pmt = pmt_ref[...]
    dx = pm[:, 0:1] - pmt[0:1, :]
    dy = pm[:, 1:2] - pmt[1:2, :]
    dz = pm[:, 2:3] - pmt[2:3, :]
    dln = jnp.sqrt(dx * dx + dy * dy + dz * dz + 1e-12)
    inv = 1.0 / (dln + 1e-9)
    ddm = dln[:, :, None] - rbfc_ref[...][None, :, :]
    efm = jnp.exp(-(ddm * ddm) / (2.0 * (CUT / RD) * (CUT / RD)))

    q = jnp.dot(hms, wq_ref[...], preferred_element_type=jnp.float32)
    k = jnp.dot(hms, wk_ref[...], preferred_element_type=jnp.float32)
    logits = lax.dot_general(q, k, (((1,), (1,)), ((), ())),
                             preferred_element_type=jnp.float32) * 0.125
    mx1 = jnp.max(logits, axis=1, keepdims=True)
    ex1 = jnp.exp(logits - mx1)
    attn = ex1 / jnp.sum(ex1, axis=1, keepdims=True)
    logits_t = lax.dot_general(k, q, (((1,), (1,)), ((), ())),
                               preferred_element_type=jnp.float32) * 0.125
    mx1t = jnp.max(logits_t, axis=1, keepdims=True)
    ex1t = jnp.exp(logits_t - mx1t)
    attnt = ex1t / jnp.sum(ex1t, axis=1, keepdims=True)

    av = attn * (attn > LAM).astype(jnp.float32) * (adj == 0).astype(jnp.float32)
    av_ref[...] = av
    avt = attnt * (attnt > LAM).astype(jnp.float32) * (adjt == 0).astype(jnp.float32)
    ri = lax.broadcasted_iota(jnp.int32, (M, M), 0)
    ci = lax.broadcasted_iota(jnp.int32, (M, M), 1)
    noteye = ri != ci
    vm = ((av > 0) | (avt > 0)) & noteye
    vmf = vm.astype(jnp.float32)

    wa1 = wa1_ref[...]
    a_i = jnp.dot(hms, wa1[:S, :], preferred_element_type=jnp.float32)
    b_j = jnp.dot(hms, wa1[S:2 * S, :], preferred_element_type=jnp.float32)
    cgeo = jnp.dot(efm.reshape(M * M, RD), wa1[2 * S:, :],
                   preferred_element_type=jnp.float32).reshape(M, M, S)
    tt = jnp.maximum(a_i[:, None, :] + b_j[None, :, :] + cgeo + ba1_ref[...][None, :, :], 0.0)
    sc = jnp.dot(tt.reshape(M * M, S), wa2_ref[...],
                 preferred_element_type=jnp.float32).reshape(M, M) + ba2_ref[0, 0]
    scm = jnp.where(vm, sc, -1e30)
    mx0 = jnp.max(scm, axis=0, keepdims=True)
    ex0 = jnp.where(vm, jnp.exp(scm - mx0), 0.0)
    dn0 = jnp.sum(ex0, axis=0, keepdims=True)
    decay = ex0 / jnp.maximum(dn0, 1e-9)

    ind = (adj > 0).astype(jnp.float32)
    wpair = ind + vmf * decay
    exists = ((adj > 0) | vm).astype(jnp.float32)

    wsh = wsh_ref[...]
    wfc = wfc_ref[...]
    bfc = bfc_ref[...]

    def blk(ib, acc):
        sl = slice(ib * _MB, (ib + 1) * _MB)
        efs = efm[sl] * wpair[sl][:, :, None]
        fw = jnp.dot(efs.reshape(_MB * M, RD), wfc,
                     preferred_element_type=jnp.float32).reshape(_MB, M, D) + bfc[None, :, :]
        lden = dln[sl] + 1e-9
        shm = jnp.concatenate(
            [jnp.full((_MB, M, 1), C0, jnp.float32),
             C1 * (dx[sl] / lden)[:, :, None], C1 * (dy[sl] / lden)[:, :, None],
             C1 * (dz[sl] / lden)[:, :, None]], axis=2)
        smm = jnp.dot(shm.reshape(_MB * M, 4), wsh,
                      preferred_element_type=jnp.float32).reshape(_MB, M, D)
        msgm = hm[sl][:, None, :] * fw * smm * exists[sl][:, :, None]
        return acc + jnp.sum(msgm, axis=0)

    aggm = jnp.zeros((M, D), jnp.float32)
    for ib in range(M // _MB):
        aggm = blk(ib, aggm)
    hh_ref[...] = (jnp.dot(aggm, wout_ref[...], preferred_element_type=jnp.float32)
                   + bout_ref[...] + hm)


def _master(h_m, p_m, adjc, rbf_c, Wq, Wk, Wa1, ba1, Wa2, ba2, Wfc, bfc, Wsh, Wout, bout):
    full = lambda s: pl.BlockSpec(s, lambda: tuple(0 for _ in s))
    return pl.pallas_call(
        _master_body,
        in_specs=[
            full((M, D)), full((M, 3)), full((3, M)), full((M, M)), full((M, M)),
            full((1, RD)),
            full((S, S)), full((S, S)), full((2 * S + RD, S)), full((1, S)),
            full((S, 1)), full((1, 1)), full((RD, D)), full((1, D)),
            full((4, D)), full((D, D)), full((1, D)),
        ],
        out_specs=[full((M, M)), full((M, D))],
        out_shape=[
            jax.ShapeDtypeStruct((M, M), jnp.float32),
            jax.ShapeDtypeStruct((M, D), jnp.float32),
        ],
    )(h_m, p_m, p_m.T, adjc, adjc.T, rbf_c.reshape(1, RD), Wq, Wk, Wa1, ba1.reshape(1, S),
      Wa2, ba2.reshape(1, 1), Wfc, bfc.reshape(1, D), Wsh, Wout,
      bout.reshape(1, D))


def _combine_body(hl_ref, m_ref, mi_ref, hh_ref, out_ref):
    i = pl.program_id(0)
    ridx = lax.broadcasted_iota(jnp.int32, (_N_TILE, M), 0) + i * _N_TILE
    onehot = (ridx == mi_ref[...]).astype(jnp.float32)
    hexp = jnp.dot(onehot, hh_ref[...], precision=lax.Precision.HIGHEST,
                   preferred_element_type=jnp.float32)
    m = m_ref[...]
    out_ref[...] = (1.0 - m) * hl_ref[...] + m * hexp


def _combine(h_local, m2, mi, hh):
    grid = (N // _N_TILE,)
    return pl.pallas_call(
        _combine_body,
        grid=grid,
        in_specs=[
            pl.BlockSpec((_N_TILE, D), lambda i: (i, 0)),
            pl.BlockSpec((_N_TILE, 1), lambda i: (i, 0)),
            pl.BlockSpec((1, M), lambda i: (0, 0)),
            pl.BlockSpec((M, D), lambda i: (0, 0)),
        ],
        out_specs=pl.BlockSpec((_N_TILE, D), lambda i: (i, 0)),
        out_shape=jax.ShapeDtypeStruct((N, D), jnp.float32),
    )(h_local, m2, mi.reshape(1, M), hh)


def kernel(h, pos, edge_index, Wfc, bfc, Wsh, Wout, bout, Wm1, bm1, Wm2, bm2,
           Wq, Wk, Wa1, ba1, Wa2, ba2):
    src = edge_index[0]
    dst = edge_index[1]
    srcp = jnp.pad(src, (0, EPAD - E))
    dstp = jnp.pad(dst, (0, EPAD - E), constant_values=N)
    pos_pad = jnp.pad(pos, ((0, 8), (0, 0)))

    rbf_c = jnp.linspace(0.0, CUT, RD)
    vecflat, hsrc = _efront(pos_pad, srcp, dstp, h)
    vecln = vecflat.reshape(EPAD, 4)
    msg = _wmsg(vecln, hsrc, rbf_c, Wfc, bfc, Wsh)
    zeros_half = jnp.zeros((NPAD, _HD), jnp.float32)
    agg3 = _escatter(dstp, msg, zeros_half)
    h_local, m2 = _nlocal(agg3, h, Wout, bout, Wm1, bm1, Wm2, bm2)
    m = m2[:, 0]

    _, mi = jax.lax.top_k(m, M)
    mi = jnp.sort(mi)
    mi_pad = jnp.pad(mi, (0, ADJW - M), constant_values=N + 8)
    posw = jnp.pad(pos_pad, ((0, 0), (0, 13)))
    adjflat, hm, pm = _eadj(srcp, dstp, mi_pad, h_local, posw)
    adj2 = adjflat.reshape(2, ADJW, ADJW)
    adjc = adj2[0, :M, :M] + adj2[1, :M, :M]
    h_m = hm[:M]
    p_m = pm[:M, :3]

    Av, hh = _master(h_m, p_m, adjc, rbf_c, Wq, Wk, Wa1, ba1, Wa2, ba2,
                     Wfc, bfc, Wsh, Wout, bout)
    h_final = _combine(h_local, m2, mi, hh)
    return (h_final, pos, Av, m)


# speedup vs baseline: 1.0025x; 1.0025x over previous
import jax
import jax.numpy as jnp
from jax import lax
from jax.experimental import pallas as pl
from jax.experimental.pallas import tpu as pltpu
from jax.experimental.pallas import tpu_sc as plsc

N = 10000
D = 128
S = 64
RD = 32
M = 200
LAM = 0.005
CUT = 6.0
E = 160000

NC = 2
NS = 16
NW = NC * NS

EPAD = 160256
EW = EPAD // NW
NPAD = 10112
NROW = NPAD // NS

_HD = D // 2

C0 = 0.28209479177387814
C1 = 0.4886025119029199


def _iota16():
    return lax.iota(jnp.int32, 16)


_EF_K = 512
_EF_FULL = EW // _EF_K
_EF_TAIL = EW - _EF_FULL * _EF_K


def _efront_body(pos_hbm, src_hbm, dst_hbm, h_hbm, vec_hbm, hsrc_hbm,
                 pos_v, src_v, dst_v, vec_v, rows_v, sem):
    wid = lax.axis_index("s") * NC + lax.axis_index("c")
    base = wid * EW
    pltpu.sync_copy(pos_hbm, pos_v)

    def chunk(start, k):
        pltpu.sync_copy(src_hbm.at[pl.ds(start, k)], src_v.at[pl.ds(0, k)])
        pltpu.sync_copy(dst_hbm.at[pl.ds(start, k)], dst_v.at[pl.ds(0, k)])
        cp = pltpu.async_copy(h_hbm.at[src_v], rows_v, sem)

        def inner(i, _):
            sv = src_v[pl.ds(i * 16, 16)] * 3
            dv = dst_v[pl.ds(i * 16, 16)] * 3
            lanes = (i * 16 + _iota16()) * 4
            acc = jnp.full((16,), 1e-12, jnp.float32)
            for comp in range(3):
                c16 = jnp.full((16,), comp, jnp.int32)
                ps = plsc.load_gather(pos_v, [sv + c16])
                pd = plsc.load_gather(pos_v, [dv + c16])
                vc = ps - pd
                acc = acc + vc * vc
                plsc.store_scatter(vec_v, [lanes + c16], vc)
            plsc.store_scatter(vec_v, [lanes + jnp.full((16,), 3, jnp.int32)], acc)
            return _

        lax.fori_loop(0, k // 16, inner, None)
        cp.wait()
        pltpu.sync_copy(rows_v.at[pl.ds(0, k), :], hsrc_hbm.at[pl.ds(start, k), :])
        pltpu.sync_copy(vec_v.at[pl.ds(0, k * 4)], vec_hbm.at[pl.ds(start * 4, k * 4)])

    def full_chunk(t, _):
        chunk(base + t * _EF_K, _EF_K)
        return _

    lax.fori_loop(0, _EF_FULL, full_chunk, None)
    chunk(base + _EF_FULL * _EF_K, _EF_TAIL)


def _efront(pos_pad, srcp, dstp, h):
    mesh = plsc.VectorSubcoreMesh(core_axis_name="c", subcore_axis_name="s")
    return pl.kernel(
        _efront_body,
        out_type=(
            jax.ShapeDtypeStruct((EPAD * 4,), jnp.float32),
            jax.ShapeDtypeStruct((EPAD, D), jnp.float32),
        ),
        mesh=mesh,
        scratch_types=[
            pltpu.VMEM(((N + 8) * 3,), jnp.float32),
            pltpu.VMEM((_EF_K,), jnp.int32),
            pltpu.VMEM((_EF_K,), jnp.int32),
            pltpu.VMEM((_EF_K * 4,), jnp.float32),
            pltpu.VMEM((_EF_K, D), jnp.float32),
            pltpu.SemaphoreType.DMA,
        ],
        compiler_params=pltpu.CompilerParams(needs_layout_passes=False, use_tc_tiling_on_sc=False),
    )(pos_pad.reshape((N + 8) * 3), srcp, dstp, h)


_W_TILE = 2048


def _wmsg_body(vec_ref, hsrc_ref, rbfc_ref, wfc_ref, bfc_ref, wsh_ref, out_ref):
    vec = vec_ref[...]
    x = vec[:, 0:1]
    y = vec[:, 1:2]
    z = vec[:, 2:3]
    ln = jnp.sqrt(vec[:, 3:4])
    lden = ln + 1e-9
    dd = ln - rbfc_ref[...]
    ef = jnp.exp(-(dd * dd) / (2.0 * (CUT / RD) * (CUT / RD)))
    w = jnp.dot(ef, wfc_ref[...], preferred_element_type=jnp.float32) + bfc_ref[...]
    sh = jnp.concatenate(
        [jnp.full((_W_TILE, 1), C0, jnp.float32), C1 * (x / lden), C1 * (y / lden),
         C1 * (z / lden)], axis=1)
    sm = jnp.dot(sh, wsh_ref[...], preferred_element_type=jnp.float32)
    res = hsrc_ref[...] * w * sm
    out_ref[0] = res[:, :_HD]
    out_ref[1] = res[:, _HD:]


def _wmsg(vecln, hsrc, rbf_c, Wfc, bfc, Wsh):
    grid = (pl.cdiv(EPAD, _W_TILE),)
    return pl.pallas_call(
        _wmsg_body,
        grid=grid,
        in_specs=[
            pl.BlockSpec((_W_TILE, 4), lambda i: (i, 0)),
            pl.BlockSpec((_W_TILE, D), lambda i: (i, 0)),
            pl.BlockSpec((1, RD), lambda i: (0, 0)),
            pl.BlockSpec((RD, D), lambda i: (0, 0)),
            pl.BlockSpec((1, D), lambda i: (0, 0)),
            pl.BlockSpec((4, D), lambda i: (0, 0)),
        ],
        out_specs=pl.BlockSpec((2, _W_TILE, _HD), lambda i: (0, i, 0)),
        out_shape=jax.ShapeDtypeStruct((2, EPAD, _HD), jnp.float32),
    )(vecln, hsrc, rbf_c.reshape(1, RD), Wfc, bfc.reshape(1, D), Wsh)


_ES_K = 512
_ES_PER = EPAD // NS
_ES_FULL = _ES_PER // _ES_K
_ES_TAIL = _ES_PER - _ES_FULL * _ES_K


def _escatter_body(dst_hbm, msg_hbm, zeros_hbm, out_hbm,
                   agg_sh, dst_v, rows_v, dst_vt, rows_vt):
    c = lax.axis_index("c")
    s = lax.axis_index("s")
    pltpu.sync_copy(zeros_hbm.at[pl.ds(s * NROW, NROW), :],
                    agg_sh.at[pl.ds(s * NROW, NROW), :])
    plsc.subcore_barrier()
    base = s * _ES_PER

    def full_chunk(t, _):
        start = base + t * _ES_K
        pltpu.sync_copy(dst_hbm.at[pl.ds(start, _ES_K)], dst_v)
        pltpu.sync_copy(msg_hbm.at[c, pl.ds(start, _ES_K), :], rows_v)
        pltpu.sync_copy(rows_v, agg_sh.at[dst_v], add=True)
        return _

    lax.fori_loop(0, _ES_FULL, full_chunk, None)
    start = base + _ES_FULL * _ES_K
    pltpu.sync_copy(dst_hbm.at[pl.ds(start, _ES_TAIL)], dst_vt)
    pltpu.sync_copy(msg_hbm.at[c, pl.ds(start, _ES_TAIL), :], rows_vt)
    pltpu.sync_copy(rows_vt, agg_sh.at[dst_vt], add=True)
    plsc.subcore_barrier()
    pltpu.sync_copy(agg_sh.at[pl.ds(s * NROW, NROW), :],
                    out_hbm.at[c, pl.ds(s * NROW, NROW), :])


def _escatter(dstp, msg, zeros_half):
    mesh = plsc.VectorSubcoreMesh(core_axis_name="c", subcore_axis_name="s")
    return pl.kernel(
        _escatter_body,
        out_type=jax.ShapeDtypeStruct((2, NPAD, _HD), jnp.float32),
        mesh=mesh,
        scratch_types=[
            pltpu.VMEM_SHARED((NPAD, _HD), jnp.float32),
            pltpu.VMEM((_ES_K,), jnp.int32),
            pltpu.VMEM((_ES_K, _HD), jnp.float32),
            pltpu.VMEM((_ES_TAIL,), jnp.int32),
            pltpu.VMEM((_ES_TAIL, _HD), jnp.float32),
        ],
        compiler_params=pltpu.CompilerParams(needs_layout_passes=False, use_tc_tiling_on_sc=False),
    )(dstp, msg, zeros_half)


ADJW = 208
ADJSZ = ADJW * ADJW
_AJ_SLAB = ADJSZ // NS
_AJ_K = 512
_AJ_FULL = EW // _AJ_K
_AJ_TAIL = EW - _AJ_FULL * _AJ_K
NRANK = NPAD


def _eadj_body(src_hbm, dst_hbm, mi_hbm, hl_hbm, posw_hbm,
               adj_hbm, hm_hbm, pm_hbm,
               rank_v, src_v, dst_v, adj_v, acc_v, mrg_v, mi_v, mi_g, hm_v, pm_v,
               stage_sh, sem):
    c = lax.axis_index("c")
    s = lax.axis_index("s")
    wid = s * NC + c
    pltpu.sync_copy(mi_hbm, mi_v)

    neg1 = jnp.full((16,), -1, jnp.int32)

    def initrank(i, _):
        rank_v[pl.ds(i * 16, 16)] = neg1
        return _

    lax.fori_loop(0, NRANK // 16, initrank, None)

    zero16 = jnp.zeros((16,), jnp.float32)

    def initadj(i, _):
        adj_v[pl.ds(i * 16, 16)] = zero16
        return _

    lax.fori_loop(0, ADJSZ // 16, initadj, None)

    def setrank(i, _):
        mi16 = mi_v[pl.ds(i * 16, 16)]
        plsc.store_scatter(rank_v, [mi16], i * 16 + _iota16())
        return _

    lax.fori_loop(0, ADJW // 16, setrank, None)

    base = wid * EW
    ones16 = jnp.ones((16,), jnp.float32)

    def chunk(start, k):
        pltpu.sync_copy(src_hbm.at[pl.ds(start, k)], src_v.at[pl.ds(0, k)])
        pltpu.sync_copy(dst_hbm.at[pl.ds(start, k)], dst_v.at[pl.ds(0, k)])

        def inner(i, _):
            sv = src_v[pl.ds(i * 16, 16)]
            dv = dst_v[pl.ds(i * 16, 16)]
            rs = plsc.load_gather(rank_v, [sv])
            rd = plsc.load_gather(rank_v, [dv])
            valid = (rs >= 0) & (rd >= 0)
            idx = jnp.where(valid, rs * ADJW + rd, 0)
            plsc.store_scatter(adj_v, [idx], ones16, mask=valid)
            return _

        lax.fori_loop(0, k // 16, inner, None)

    def full_chunk(t, _):
        chunk(base + t * _AJ_K, _AJ_K)
        return _

    lax.fori_loop(0, _AJ_FULL, full_chunk, None)
    chunk(base + _AJ_FULL * _AJ_K, _AJ_TAIL)

    pltpu.sync_copy(adj_v, stage_sh.at[pl.ds(s * ADJSZ, ADJSZ)])
    plsc.subcore_barrier()

    pltpu.sync_copy(stage_sh.at[pl.ds(s * _AJ_SLAB, _AJ_SLAB)], acc_v)
    for t in range(1, NS):
        pltpu.sync_copy(stage_sh.at[pl.ds(t * ADJSZ + s * _AJ_SLAB, _AJ_SLAB)], mrg_v)

        def madd(i, _):
            acc_v[pl.ds(i * 16, 16)] = acc_v[pl.ds(i * 16, 16)] + mrg_v[pl.ds(i * 16, 16)]
            return _

        lax.fori_loop(0, _AJ_SLAB // 16, madd, None)
    pltpu.sync_copy(acc_v, adj_hbm.at[c, pl.ds(s * _AJ_SLAB, _AJ_SLAB)])

    @pl.when((c == 0) & (s == 0))
    def _():
        pltpu.sync_copy(mi_hbm.at[pl.ds(0, ADJW // 2)], mi_g)
        pltpu.async_copy(hl_hbm.at[mi_g], hm_v, sem).wait()
        pltpu.sync_copy(hm_v, hm_hbm.at[pl.ds(0, ADJW // 2), :])
        pltpu.sync_copy(mi_hbm.at[pl.ds(ADJW // 2, ADJW // 2)], mi_g)
        pltpu.async_copy(hl_hbm.at[mi_g], hm_v, sem).wait()
        pltpu.sync_copy(hm_v, hm_hbm.at[pl.ds(ADJW // 2, ADJW // 2), :])

    @pl.when((c == 1) & (s == 0))
    def _():
        pltpu.async_copy(posw_hbm.at[mi_v], pm_v, sem).wait()
        pltpu.sync_copy(pm_v, pm_hbm)


def _eadj(srcp, dstp, mi_pad, h_local, posw):
    mesh = plsc.VectorSubcoreMesh(core_axis_name="c", subcore_axis_name="s")
    return pl.kernel(
        _eadj_body,
        out_type=(
            jax.ShapeDtypeStruct((2, ADJSZ), jnp.float32),
            jax.ShapeDtypeStruct((ADJW, D), jnp.float32),
            jax.ShapeDtypeStruct((ADJW, 16), jnp.float32),
        ),
        mesh=mesh,
        scratch_types=[
            pltpu.VMEM((NRANK,), jnp.int32),
            pltpu.VMEM((_AJ_K,), jnp.int32),
            pltpu.VMEM((_AJ_K,), jnp.int32),
            pltpu.VMEM((ADJSZ,), jnp.float32),
            pltpu.VMEM((_AJ_SLAB,), jnp.float32),
            pltpu.VMEM((_AJ_SLAB,), jnp.float32),
            pltpu.VMEM((ADJW,), jnp.int32),
            pltpu.VMEM((ADJW // 2,), jnp.int32),
            pltpu.VMEM((ADJW // 2, D), jnp.float32),
            pltpu.VMEM((ADJW, 16), jnp.float32),
            pltpu.VMEM_SHARED((NS * ADJSZ,), jnp.float32),
            pltpu.SemaphoreType.DMA,
        ],
        compiler_params=pltpu.CompilerParams(needs_layout_passes=False, use_tc_tiling_on_sc=False),
    )(srcp, dstp, mi_pad, h_local, posw)


_N_TILE = 400


def _nlocal_body(agg0_ref, agg1_ref, h_ref, wout_ref, bout_ref,
                 wm1_ref, bm1_ref, wm2_ref, bm2_ref, hl_ref, m_ref):
    agg = jnp.concatenate([agg0_ref[0], agg1_ref[0]], axis=1)
    hl = (jnp.dot(agg, wout_ref[...], preferred_element_type=jnp.float32)
          + bout_ref[...] + h_ref[...])
    hl_ref[...] = hl
    t = jnp.maximum(jnp.dot(hl[:, :S], wm1_ref[...], preferred_element_type=jnp.float32)
                    + bm1_ref[...], 0.0)
    logit = jnp.dot(t, wm2_ref[...], preferred_element_type=jnp.float32) + bm2_ref[...]
    m_ref[...] = 1.0 / (1.0 + jnp.exp(-logit))


def _nlocal(agg3, h, Wout, bout, Wm1, bm1, Wm2, bm2):
    grid = (N // _N_TILE,)
    return pl.pallas_call(
        _nlocal_body,
        grid=grid,
        in_specs=[
            pl.BlockSpec((1, _N_TILE, _HD), lambda i: (0, i, 0)),
            pl.BlockSpec((1, _N_TILE, _HD), lambda i: (1, i, 0)),
            pl.BlockSpec((_N_TILE, D), lambda i: (i, 0)),
            pl.BlockSpec((D, D), lambda i: (0, 0)),
            pl.BlockSpec((1, D), lambda i: (0, 0)),
            pl.BlockSpec((S, D), lambda i: (0, 0)),
            pl.BlockSpec((1, D), lambda i: (0, 0)),
            pl.BlockSpec((D, 1), lambda i: (0, 0)),
            pl.BlockSpec((1, 1), lambda i: (0, 0)),
        ],
        out_specs=[
            pl.BlockSpec((_N_TILE, D), lambda i: (i, 0)),
            pl.BlockSpec((_N_TILE, 1), lambda i: (i, 0)),
        ],
        out_shape=[
            jax.ShapeDtypeStruct((N, D), jnp.float32),
            jax.ShapeDtypeStruct((N, 1), jnp.float32),
        ],
    )(agg3, agg3, h, Wout, bout.reshape(1, D), Wm1, bm1.reshape(1, D),
      Wm2, bm2.reshape(1, 1))


_MB = 8


def _master_body(hm_ref, pm_ref, pmt_ref, adj_ref, adjt_ref, rbfc_ref, wq_ref, wk_ref,
                 wa1_ref, ba1_ref, wa2_ref, ba2_ref, wfc_ref, bfc_ref,
                 wsh_ref, wout_ref, bout_ref, av_ref, hh_ref):
    hm = hm_ref[...]
    hms = hm[:, :S]
    adj = adj_ref[...]
    adjt = adjt_ref[...]

    pm = pm_ref[...]
    pmt = pmt_ref[...]
    dx = pm[:, 0:1] - pmt[0:1, :]
    dy = pm[:, 1:2] - pmt[1:2, :]
    dz = pm[:, 2:3] - pmt[2:3, :]
    dln = jnp.sqrt(dx * dx + dy * dy + dz * dz + 1e-12)
    ddm = dln[:, :, None] - rbfc_ref[...][None, :, :]
    efm = jnp.exp(-(ddm * ddm) / (2.0 * (CUT / RD) * (CUT / RD)))

    q = jnp.dot(hms, wq_ref[...], preferred_element_type=jnp.float32)
    k = jnp.dot(hms, wk_ref[...], preferred_element_type=jnp.float32)
    logits = lax.dot_general(q, k, (((1,), (1,)), ((), ())),
                             preferred_element_type=jnp.float32) * 0.125
    mx1 = jnp.max(logits, axis=1, keepdims=True)
    ex1 = jnp.exp(logits - mx1)
    attn = ex1 / jnp.sum(ex1, axis=1, keepdims=True)
    logits_t = lax.dot_general(k, q, (((1,), (1,)), ((), ())),
                               preferred_element_type=jnp.float32) * 0.125
    mx1t = jnp.max(logits_t, axis=1, keepdims=True)
    ex1t = jnp.exp(logits_t - mx1t)
    attnt = ex1t / jnp.sum(ex1t, axis=1, keepdims=True)

    av = attn * (attn > LAM).astype(jnp.float32) * (adj == 0).astype(jnp.float32)
    av_ref[...] = av
    avt = attnt * (attnt > LAM).astype(jnp.float32) * (adjt == 0).astype(jnp.float32)
    ri = lax.broadcasted_iota(jnp.int32, (M, M), 0)
    ci = lax.broadcasted_iota(jnp.int32, (M, M), 1)
    noteye = ri != ci
    vm = ((av > 0) | (avt > 0)) & noteye
    vmf = vm.astype(jnp.float32)

    wa1 = wa1_ref[...]
    a_i = jnp.dot(hms, wa1[:S, :], preferred_element_type=jnp.float32)
    b_j = jnp.dot(hms, wa1[S:2 * S, :], preferred_element_type=jnp.float32)
    cgeo = jnp.dot(efm.reshape(M * M, RD), wa1[2 * S:, :],
                   preferred_element_type=jnp.float32).reshape(M, M, S)
    tt = jnp.maximum(a_i[:, None, :] + b_j[None, :, :] + cgeo + ba1_ref[...][None, :, :], 0.0)
    sc = jnp.dot(tt.reshape(M * M, S), wa2_ref[...],
                 preferred_element_type=jnp.float32).reshape(M, M) + ba2_ref[0, 0]
    scm = jnp.where(vm, sc, -1e30)
    mx0 = jnp.max(scm, axis=0, keepdims=True)
    ex0 = jnp.where(vm, jnp.exp(scm - mx0), 0.0)
    dn0 = jnp.sum(ex0, axis=0, keepdims=True)
    decay = ex0 / jnp.maximum(dn0, 1e-9)

    ind = (adj > 0).astype(jnp.float32)
    wpair = ind + vmf * decay
    exists = ((adj > 0) | vm).astype(jnp.float32)

    wsh = wsh_ref[...]
    wfc = wfc_ref[...]
    bfc = bfc_ref[...]

    def blk(ib, acc):
        sl = slice(ib * _MB, (ib + 1) * _MB)
        efs = efm[sl] * wpair[sl][:, :, None]
        fw = jnp.dot(efs.reshape(_MB * M, RD), wfc,
                     preferred_element_type=jnp.float32).reshape(_MB, M, D) + bfc[None, :, :]
        lden = dln[sl] + 1e-9
        shm = jnp.concatenate(
            [jnp.full((_MB, M, 1), C0, jnp.float32),
             C1 * (dx[sl] / lden)[:, :, None], C1 * (dy[sl] / lden)[:, :, None],
             C1 * (dz[sl] / lden)[:, :, None]], axis=2)
        smm = jnp.dot(shm.reshape(_MB * M, 4), wsh,
                      preferred_element_type=jnp.float32).reshape(_MB, M, D)
        msgm = hm[sl][:, None, :] * fw * smm * exists[sl][:, :, None]
        return acc + jnp.sum(msgm, axis=0)

    aggm = jnp.zeros((M, D), jnp.float32)
    for ib in range(M // _MB):
        aggm = blk(ib, aggm)
    hh_ref[...] = (jnp.dot(aggm, wout_ref[...], preferred_element_type=jnp.float32)
                   + bout_ref[...] + hm)


def _master(h_m, p_m, adjc, rbf_c, Wq, Wk, Wa1, ba1, Wa2, ba2, Wfc, bfc, Wsh, Wout, bout):
    full = lambda s: pl.BlockSpec(s, lambda: tuple(0 for _ in s))
    return pl.pallas_call(
        _master_body,
        in_specs=[
            full((M, D)), full((M, 3)), full((3, M)), full((M, M)), full((M, M)),
            full((1, RD)),
            full((S, S)), full((S, S)), full((2 * S + RD, S)), full((1, S)),
            full((S, 1)), full((1, 1)), full((RD, D)), full((1, D)),
            full((4, D)), full((D, D)), full((1, D)),
        ],
        out_specs=[full((M, M)), full((M, D))],
        out_shape=[
            jax.ShapeDtypeStruct((M, M), jnp.float32),
            jax.ShapeDtypeStruct((M, D), jnp.float32),
        ],
    )(h_m, p_m, p_m.T, adjc, adjc.T, rbf_c.reshape(1, RD), Wq, Wk, Wa1, ba1.reshape(1, S),
      Wa2, ba2.reshape(1, 1), Wfc, bfc.reshape(1, D), Wsh, Wout,
      bout.reshape(1, D))


def _combine_body(hl_ref, m_ref, mi_ref, hh_ref, out_ref):
    i = pl.program_id(0)
    ridx = lax.broadcasted_iota(jnp.int32, (_N_TILE, M), 0) + i * _N_TILE
    onehot = (ridx == mi_ref[...]).astype(jnp.float32)
    hexp = jnp.dot(onehot, hh_ref[...], precision=lax.Precision.HIGHEST,
                   preferred_element_type=jnp.float32)
    m = m_ref[...]
    out_ref[...] = (1.0 - m) * hl_ref[...] + m * hexp


def _combine(h_local, m2, mi, hh):
    grid = (N // _N_TILE,)
    return pl.pallas_call(
        _combine_body,
        grid=grid,
        in_specs=[
            pl.BlockSpec((_N_TILE, D), lambda i: (i, 0)),
            pl.BlockSpec((_N_TILE, 1), lambda i: (i, 0)),
            pl.BlockSpec((1, M), lambda i: (0, 0)),
            pl.BlockSpec((M, D), lambda i: (0, 0)),
        ],
        out_specs=pl.BlockSpec((_N_TILE, D), lambda i: (i, 0)),
        out_shape=jax.ShapeDtypeStruct((N, D), jnp.float32),
    )(h_local, m2, mi.reshape(1, M), hh)


def kernel(h, pos, edge_index, Wfc, bfc, Wsh, Wout, bout, Wm1, bm1, Wm2, bm2,
           Wq, Wk, Wa1, ba1, Wa2, ba2):
    src = edge_index[0]
    dst = edge_index[1]
    srcp = jnp.pad(src, (0, EPAD - E))
    dstp = jnp.pad(dst, (0, EPAD - E), constant_values=N)
    pos_pad = jnp.pad(pos, ((0, 8), (0, 0)))

    rbf_c = jnp.linspace(0.0, CUT, RD)
    vecflat, hsrc = _efront(pos_pad, srcp, dstp, h)
    vecln = vecflat.reshape(EPAD, 4)
    msg = _wmsg(vecln, hsrc, rbf_c, Wfc, bfc, Wsh)
    zeros_half = jnp.zeros((NPAD, _HD), jnp.float32)
    agg3 = _escatter(dstp, msg, zeros_half)
    h_local, m2 = _nlocal(agg3, h, Wout, bout, Wm1, bm1, Wm2, bm2)
    m = m2[:, 0]

    _, mi = jax.lax.top_k(m, M)
    mi = jnp.sort(mi)
    mi_pad = jnp.pad(mi, (0, ADJW - M), constant_values=N + 8)
    posw = jnp.pad(pos_pad, ((0, 0), (0, 13)))
    adjflat, hm, pm = _eadj(srcp, dstp, mi_pad, h_local, posw)
    adj2 = adjflat.reshape(2, ADJW, ADJW)
    adjc = adj2[0, :M, :M] + adj2[1, :M, :M]
    h_m = hm[:M]
    p_m = pm[:M, :3]

    Av, hh = _master(h_m, p_m, adjc, rbf_c, Wq, Wk, Wa1, ba1, Wa2, ba2,
                     Wfc, bfc, Wsh, Wout, bout)
    h_final = _combine(h_local, m2, mi, hh)
    return (h_final, pos, Av, m)
